# baseline (device time: 45496 ns/iter reference)
import jax
import jax.numpy as jnp
from jax import lax
from jax.experimental import pallas as pl
from jax.experimental.pallas import tpu as pltpu

N_DEV = 4


def _scan8(y):
    row = lax.broadcasted_iota(jnp.int32, y.shape, 1)
    for s in (1, 2, 4):
        shifted = pltpu.roll(y, s, 1)
        y = y * jnp.where(row < s, jnp.float32(1.0), shifted)
    return y


def _exclusive8(y):
    row = lax.broadcasted_iota(jnp.int32, y.shape, 1)
    return jnp.where(row < 1, jnp.float32(1.0), pltpu.roll(y, 1, 1))


def kernel(x):
    m, n = x.shape

    def body(x_ref, out_ref, totals_ref, send_sems, recv_sems):
        my = lax.axis_index("i")

        barrier_sem = pltpu.get_barrier_semaphore()
        for off in range(1, N_DEV):
            pl.semaphore_signal(
                barrier_sem,
                inc=1,
                device_id=((my + off) % N_DEV,),
                device_id_type=pl.DeviceIdType.MESH,
            )
        pl.semaphore_wait(barrier_sem, N_DEV - 1)

        s0 = _scan8(x_ref[:, :].reshape(m // 8, 8, n))
        t0 = s0[:, 7, :]
        out_ref[:, :] = s0.reshape(m, n)
        s1 = _scan8(t0.reshape(m // 64, 8, n))
        s2 = _scan8(s1[:, 7, :].reshape(m // 512, 8, n))
        s3 = _scan8(s2[:, 7, :].reshape(1, 8, n))
        totals_ref[pl.ds(my, 1), :] = s3[:, 7, :]

        sends = []
        for k in range(N_DEV - 1):
            rdma = pltpu.make_async_remote_copy(
                src_ref=totals_ref.at[pl.ds(my, 1)],
                dst_ref=totals_ref.at[pl.ds(my, 1)],
                send_sem=send_sems.at[k],
                recv_sem=recv_sems.at[k],
                device_id=((my + k + 1) % N_DEV,),
                device_id_type=pl.DeviceIdType.MESH,
            )
            rdma.start()
            sends.append(rdma)

        for k in range(N_DEV - 1):
            src_row = (my - 1 - k) % N_DEV
            recv = pltpu.make_async_remote_copy(
                src_ref=totals_ref.at[pl.ds(src_row, 1)],
                dst_ref=totals_ref.at[pl.ds(src_row, 1)],
                send_sem=send_sems.at[k],
                recv_sem=recv_sems.at[k],
                device_id=(my,),
                device_id_type=pl.DeviceIdType.MESH,
            )
            recv.wait_recv()
        for rdma in sends:
            rdma.wait_send()

        totals = totals_ref[:, :]
        rid = lax.broadcasted_iota(jnp.int32, (N_DEV, n), 0)
        factors = jnp.where(rid < my, totals, jnp.ones_like(totals))
        prefix = factors[0] * factors[1] * factors[2] * factors[3]

        p3 = (_exclusive8(s3) * prefix.reshape(1, 1, n)).reshape(8, n)
        p2 = (_exclusive8(s2) * p3.reshape(m // 512, 1, n)).reshape(m // 64, n)
        p1 = (_exclusive8(s1) * p2.reshape(m // 64, 1, n)).reshape(m // 8, n)
        out = out_ref[:, :].reshape(m // 8, 8, n) * p1.reshape(m // 8, 1, n)
        out_ref[:, :] = out.reshape(m, n)

    return pl.pallas_call(
        body,
        out_shape=jax.ShapeDtypeStruct((m, n), jnp.float32),
        in_specs=[pl.BlockSpec(memory_space=pltpu.VMEM)],
        out_specs=pl.BlockSpec(memory_space=pltpu.VMEM),
        scratch_shapes=[
            pltpu.VMEM((N_DEV, n), jnp.float32),
            pltpu.SemaphoreType.DMA((N_DEV - 1,)),
            pltpu.SemaphoreType.DMA((N_DEV - 1,)),
        ],
        compiler_params=pltpu.CompilerParams(
            collective_id=0, vmem_limit_bytes=100 * 1024 * 1024
        ),
    )(x)
